# baseline (speedup 1.0000x reference)
import functools

import jax
import jax.numpy as jnp
from jax.experimental import pallas as pl
from jax.experimental.pallas import tpu as pltpu


def _fused_kernel(x_ref, y_ref, row_ref, out_ref, loss_ref, acc_ref):
    i = pl.program_id(0)
    n = pl.num_programs(0)
    row = row_ref[...]
    out_ref[...] = row

    m = jnp.max(row)
    s = jnp.sum(jnp.exp(row - m))
    lse = m + jnp.log(s)

    yi = y_ref[i]
    lane = jax.lax.broadcasted_iota(jnp.int32, row.shape, 2)
    picked = jnp.sum(jnp.where(lane == yi, row, 0.0))

    val = lse - picked

    @pl.when(i == 0)
    def _():
        acc_ref[0] = 0.0
        acc_ref[1] = 0.0

    yk = val - acc_ref[1]
    t = acc_ref[0] + yk
    acc_ref[1] = (t - acc_ref[0]) - yk
    acc_ref[0] = t

    @pl.when(i == n - 1)
    def _():
        loss_ref[0, 0] = acc_ref[0] / n


def kernel(x, y, table):
    B, T = x.shape
    V, C = table.shape
    N = B * T
    xf = x.reshape(N).astype(jnp.int32)
    yf = y.reshape(N).astype(jnp.int32)
    table3 = table.reshape(V, 1, C)

    grid_spec = pltpu.PrefetchScalarGridSpec(
        num_scalar_prefetch=2,
        grid=(N,),
        in_specs=[
            pl.BlockSpec((1, 1, C), lambda i, xr, yr: (xr[i], 0, 0)),
        ],
        out_specs=[
            pl.BlockSpec((1, 1, C), lambda i, xr, yr: (i, 0, 0)),
            pl.BlockSpec(memory_space=pltpu.SMEM),
        ],
        scratch_shapes=[pltpu.SMEM((2,), jnp.float32)],
    )

    logits_flat, loss = pl.pallas_call(
        _fused_kernel,
        grid_spec=grid_spec,
        out_shape=[
            jax.ShapeDtypeStruct((N, 1, C), table.dtype),
            jax.ShapeDtypeStruct((1, 1), jnp.float32),
        ],
    )(xf, yf, table3)

    return (logits_flat.reshape(B, T, C), loss.reshape(()))


# speedup vs baseline: 4.9659x; 4.9659x over previous
import dataclasses

import jax
import jax.numpy as jnp
from jax import lax
from jax.experimental import pallas as pl
from jax.experimental.pallas import tpu as pltpu
from jax.experimental.pallas import tpu_sc as plsc

_NC = 2
_NS = 16
_NW = _NC * _NS
_LANES = 16


def _lse_scan_kernel(tab_ref, lse_ref):
    blk = tab_ref[...]
    m = jnp.max(blk, axis=1, keepdims=True)
    s = jnp.sum(jnp.exp(blk - m), axis=1)
    lse = m[:, 0] + jnp.log(s)
    lse_ref[...] = lse.reshape(1, 1, -1)


def _table_lse(table, row_block=256):
    V, C = table.shape
    out = pl.pallas_call(
        _lse_scan_kernel,
        grid=(V // row_block,),
        in_specs=[pl.BlockSpec((row_block, C), lambda i: (i, 0))],
        out_specs=pl.BlockSpec((1, 1, row_block), lambda i: (i, 0, 0)),
        out_shape=jax.ShapeDtypeStruct((V // row_block, 1, row_block),
                                       jnp.float32),
        compiler_params=pltpu.CompilerParams(
            dimension_semantics=("parallel",)),
    )(table)
    return out.reshape(V)


def _sc_gather(table2, xg):
    R, D = table2.shape
    Bq = xg.shape[0]
    bpw = Bq // _NW
    ch = 8
    nch = bpw // ch

    mesh = plsc.VectorSubcoreMesh(core_axis_name="c", subcore_axis_name="s")

    @pl.kernel(
        out_type=jax.ShapeDtypeStruct((Bq, D), jnp.float32),
        mesh=mesh,
        scratch_types=[
            pltpu.VMEM((bpw,), jnp.int32),
            pltpu.VMEM((ch, D), jnp.float32),
            pltpu.VMEM((ch, D), jnp.float32),
            pltpu.SemaphoreType.DMA,
            pltpu.SemaphoreType.DMA,
            pltpu.SemaphoreType.DMA,
            pltpu.SemaphoreType.DMA,
        ],
    )
    def k(tab_hbm, idx_hbm, out_hbm, idx_v, rows0, rows1, gs0, gs1, ws0, ws1):
        wid = lax.axis_index("s") * _NC + lax.axis_index("c")
        base = wid * bpw
        pltpu.sync_copy(idx_hbm.at[pl.ds(base, bpw)], idx_v)

        def gst(c, rv, sem):
            pltpu.async_copy(tab_hbm.at[idx_v.at[pl.ds(c * ch, ch)]], rv, sem)

        def gwt(c, rv, sem):
            pltpu.make_async_copy(
                tab_hbm.at[idx_v.at[pl.ds(c * ch, ch)]], rv, sem).wait()

        def wst(c, rv, sem):
            pltpu.async_copy(rv, out_hbm.at[pl.ds(base + c * ch, ch)], sem)

        def wwt(c, rv, sem):
            pltpu.make_async_copy(
                rv, out_hbm.at[pl.ds(base + c * ch, ch)], sem).wait()

        gst(0, rows0, gs0)
        gst(1, rows1, gs1)

        @pl.loop(0, nch, step=2)
        def _(g):
            for b, rv, gs, ws in ((0, rows0, gs0, ws0), (1, rows1, gs1, ws1)):
                c = g + b
                gwt(c, rv, gs)
                wst(c, rv, ws)

                @pl.when(c + 2 < nch)
                def _():
                    wwt(c, rv, ws)
                    gst(c + 2, rv, gs)

        wwt(nch - 2, rows0, ws0)
        wwt(nch - 1, rows1, ws1)

    return k(table2, xg)


def _sc_combine(lse_all, xf, blk, ym, table16):
    N = xf.shape[0]
    V = lse_all.shape[0]
    bpw = N // _NW
    ngrp = bpw // _LANES

    mesh = plsc.VectorSubcoreMesh(core_axis_name="c", subcore_axis_name="s")
    cp = pltpu.CompilerParams()
    if "needs_layout_passes" in pltpu.CompilerParams.__dataclass_fields__:
        cp = dataclasses.replace(cp, needs_layout_passes=False)

    @pl.kernel(
        out_type=jax.ShapeDtypeStruct((_NW, _LANES), jnp.float32),
        mesh=mesh,
        compiler_params=cp,
        scratch_types=[
            pltpu.VMEM((V,), jnp.float32),
            pltpu.VMEM((bpw,), jnp.int32),
            pltpu.VMEM((bpw,), jnp.int32),
            pltpu.VMEM((bpw,), jnp.int32),
            pltpu.VMEM((bpw, 128), jnp.float32),
            pltpu.VMEM((_LANES,), jnp.float32),
            pltpu.SemaphoreType.DMA,
        ],
    )
    def k(lse_hbm, x_hbm, blk_hbm, ym_hbm, tab16_hbm, part_hbm,
          lse_v, x_v, blk_v, ym_v, pb_v, acc_v, sem):
        wid = lax.axis_index("s") * _NC + lax.axis_index("c")
        base = wid * bpw
        pltpu.sync_copy(lse_hbm, lse_v)
        pltpu.sync_copy(x_hbm.at[pl.ds(base, bpw)], x_v)
        pltpu.sync_copy(blk_hbm.at[pl.ds(base, bpw)], blk_v)
        pltpu.sync_copy(ym_hbm.at[pl.ds(base, bpw)], ym_v)
        pltpu.async_copy(tab16_hbm.at[blk_v], pb_v, sem).wait()

        acc_v[...] = jnp.zeros((_LANES,), jnp.float32)

        @pl.loop(0, ngrp)
        def _(t):
            off = t * _LANES
            xr = x_v[pl.ds(off, _LANES)]
            ymr = ym_v[pl.ds(off, _LANES)]
            lg = plsc.load_gather(lse_v, [xr])
            rowi = lax.broadcasted_iota(jnp.int32, (_LANES,), 0) + off
            pk = plsc.load_gather(pb_v, [rowi, ymr])
            acc_v[...] = acc_v[...] + lg - pk

        pltpu.sync_copy(acc_v, part_hbm.at[wid])

    return k(lse_all, xf, blk, ym, table16)


def _final_reduce(partials, n_tokens):
    def body(part_ref, loss_ref):
        loss_ref[0, 0] = jnp.sum(part_ref[...]) / n_tokens

    return pl.pallas_call(
        body,
        in_specs=[pl.BlockSpec(partials.shape, lambda: (0, 0))],
        out_specs=pl.BlockSpec(memory_space=pltpu.SMEM),
        out_shape=jax.ShapeDtypeStruct((1, 1), jnp.float32),
    )(partials)


def kernel(x, y, table):
    B, T = x.shape
    V, C = table.shape
    N = B * T

    xf = x.reshape(N).astype(jnp.int32)
    yf = y.reshape(N).astype(jnp.int32)

    lse_all = _table_lse(table)

    xg = (2 * xf[:, None]
          + jnp.arange(2, dtype=jnp.int32)[None, :]).reshape(2 * N)
    table2 = table.reshape(2 * V, C // 2)
    logits_flat = _sc_gather(table2, xg).reshape(N, C)

    blk = xf * (C // 128) + yf // 128
    ym = yf & 127
    table128 = table.reshape(V * C // 128, 128)
    partials = _sc_combine(lse_all, xf, blk, ym, table128)

    loss = _final_reduce(partials, N)
    return (logits_flat.reshape(B, T, C), loss.reshape(()))
